# baseline (speedup 1.0000x reference)
import functools

import jax
import jax.numpy as jnp
from jax import lax
from jax.experimental import pallas as pl
from jax.experimental.pallas import tpu as pltpu
from jax.experimental.pallas import tpu_sc as plsc

B = 16
L = 4096
D = 1024

NC = 2
NS = 16
LANES = 16

BPC = B // NC
NSEG = 2 * BPC
CHUNK = 64
DH = D // 2
NV = DH // LANES

_NEG = float("-inf")


def _sc_body(padded_hbm, lens_hbm, out_hbm,
             lens_v, buf0, buf1, obuf, cbuf, partials, sem0, sem1):
    c = lax.axis_index("c")
    s = lax.axis_index("s")

    pltpu.sync_copy(lens_hbm, lens_v.at[pl.ds(0, B)])

    def seg_len(j):
        return lens_v[pl.ds(BPC * c + j // 2, LANES)][0]

    neg = jnp.full((LANES,), _NEG, jnp.float32)

    for v in range(NV):
        obuf[pl.ds(v * LANES, LANES)] = neg

    def init_body(j, carry):
        pltpu.sync_copy(obuf, partials.at[j, s])
        return carry

    lax.fori_loop(0, NSEG, init_body, 0)

    def sum_body(j, tot):
        return tot + seg_len(j)

    total = lax.fori_loop(0, NSEG, sum_body, jnp.int32(0))
    u0 = (s * total) // NS
    u1 = ((s + 1) * total) // NS

    def seg_body(j, start):
        n = seg_len(j)
        b = BPC * c + j // 2
        h = j % 2
        lo = jnp.clip(u0 - start, 0, n)
        hi = jnp.clip(u1 - start, 0, n)

        @pl.when(lo < hi)
        def _process():
            c0 = lo // CHUNK
            nch = (hi + CHUNK - 1) // CHUNK - c0

            def src(ci):
                return padded_hbm.at[
                    b, pl.ds((c0 + ci) * CHUNK, CHUNK), pl.ds(h * DH, DH)
                ]

            pltpu.async_copy(src(0), buf0, sem0)

            def rows(ci, buf, accs):
                r0 = jnp.clip(lo - (c0 + ci) * CHUNK, 0, CHUNK)
                r1 = jnp.clip(hi - (c0 + ci) * CHUNK, 0, CHUNK)

                def row_body(r, accs):
                    return tuple(
                        jnp.maximum(accs[v], buf[r, pl.ds(v * LANES, LANES)])
                        for v in range(NV)
                    )

                return lax.fori_loop(r0, r1, row_body, accs)

            def pair_body(k, accs):
                ca = 2 * k
                cb = 2 * k + 1

                @pl.when(cb < nch)
                def _():
                    pltpu.async_copy(src(cb), buf1, sem1)

                pltpu.make_async_copy(src(ca), buf0, sem0).wait()
                accs = rows(ca, buf0, accs)

                @pl.when(ca + 2 < nch)
                def _():
                    pltpu.async_copy(src(ca + 2), buf0, sem0)

                @pl.when(cb < nch)
                def _():
                    pltpu.make_async_copy(src(cb), buf1, sem1).wait()

                accs = rows(cb, buf1, accs)
                return accs

            npairs = (nch + 1) // 2
            accs = lax.fori_loop(
                0, npairs, pair_body, tuple(neg for _ in range(NV))
            )

            for v in range(NV):
                obuf[pl.ds(v * LANES, LANES)] = accs[v]
            pltpu.sync_copy(obuf, partials.at[j, s])

        return start + n

    lax.fori_loop(0, NSEG, seg_body, jnp.int32(0))

    plsc.subcore_barrier()

    n_s = seg_len(s)
    b_s = BPC * c + s // 2
    h_s = s % 2
    pltpu.sync_copy(partials.at[s], cbuf)

    def comb_body(w, accs):
        return tuple(
            jnp.maximum(accs[v], cbuf[w, pl.ds(v * LANES, LANES)])
            for v in range(NV)
        )

    accs = lax.fori_loop(0, NS, comb_body, tuple(neg for _ in range(NV)))
    for v in range(NV):
        obuf[pl.ds(v * LANES, LANES)] = accs[v]

    @pl.when(n_s == 0)
    def _zeros():
        z = jnp.zeros((LANES,), jnp.float32)
        for v in range(NV):
            obuf[pl.ds(v * LANES, LANES)] = z

    pltpu.sync_copy(obuf, out_hbm.at[b_s, pl.ds(h_s * DH, DH)])


@jax.jit
def _pooled(padded, lens):
    mesh = plsc.VectorSubcoreMesh(
        core_axis_name="c", subcore_axis_name="s", num_cores=NC, num_subcores=NS
    )
    k = pl.kernel(
        _sc_body,
        out_type=jax.ShapeDtypeStruct((B, D), jnp.float32),
        mesh=mesh,
        scratch_types=[
            pltpu.VMEM((2 * B,), jnp.int32),
            pltpu.VMEM((CHUNK, DH), jnp.float32),
            pltpu.VMEM((CHUNK, DH), jnp.float32),
            pltpu.VMEM((DH,), jnp.float32),
            pltpu.VMEM((NS, DH), jnp.float32),
            pltpu.VMEM_SHARED((NSEG, NS, DH), jnp.float32),
            pltpu.SemaphoreType.DMA,
            pltpu.SemaphoreType.DMA,
        ],
    )
    return k(padded, lens)


BL = 256
NL = L // BL


def _tc_body(lens_ref, x_ref, o_ref):
    b = pl.program_id(0)
    l = pl.program_id(1)
    n = lens_ref[b]
    nb = (n + BL - 1) // BL

    @pl.when(jnp.logical_or(l == 0, l < nb))
    def _():
        def _masked(x):
            row = jax.lax.broadcasted_iota(jnp.int32, (1, BL, D), 1) + l * BL
            return jnp.max(jnp.where(row < n, x, -jnp.inf), axis=1, keepdims=True)

        def _plain(x):
            return jnp.max(x, axis=1, keepdims=True)

        m = jax.lax.cond((l + 1) * BL > n, _masked, _plain, x_ref[...])

        @pl.when(l == 0)
        def _():
            o_ref[...] = m

        @pl.when(l > 0)
        def _():
            o_ref[...] = jnp.maximum(o_ref[...], m)

    @pl.when(l == NL - 1)
    def _():
        o_ref[...] = jnp.where(n > 0, o_ref[...], 0.0)


@jax.jit
def _pooled_tc(padded, lens):
    return pl.pallas_call(
        _tc_body,
        grid_spec=pltpu.PrefetchScalarGridSpec(
            num_scalar_prefetch=1,
            grid=(B, NL),
            in_specs=[
                pl.BlockSpec(
                    (1, BL, D),
                    lambda b, l, lens: (
                        b,
                        jnp.minimum(
                            l, jnp.maximum((lens[b] + BL - 1) // BL - 1, 0)
                        ),
                        0,
                    ),
                ),
            ],
            out_specs=pl.BlockSpec((1, 1, D), lambda b, l, lens: (b, 0, 0)),
        ),
        out_shape=jax.ShapeDtypeStruct((B, 1, D), jnp.float32),
    )(lens, padded)[:, 0, :]


def kernel(sentence_embeddings_list, num_sentences, encoder_hidden_size):
    del encoder_hidden_size
    padded = sentence_embeddings_list.astype(jnp.float32)
    lens = num_sentences.astype(jnp.int32)
    return _pooled_tc(padded, lens)


# speedup vs baseline: 1.1291x; 1.1291x over previous
import functools

import jax
import jax.numpy as jnp
from jax import lax
from jax.experimental import pallas as pl
from jax.experimental.pallas import tpu as pltpu
from jax.experimental.pallas import tpu_sc as plsc

B = 16
L = 4096
D = 1024

NC = 2
NS = 16
LANES = 16

BPC = B // NC
NSEG = 2 * BPC
CHUNK = 64
DH = D // 2
NV = DH // LANES

_NEG = float("-inf")


def _sc_body(padded_hbm, lens_hbm, out_hbm,
             lens_v, buf0, buf1, obuf, cbuf, partials, sem0, sem1):
    c = lax.axis_index("c")
    s = lax.axis_index("s")

    pltpu.sync_copy(lens_hbm, lens_v.at[pl.ds(0, B)])

    def seg_len(j):
        return lens_v[pl.ds(BPC * c + j // 2, LANES)][0]

    neg = jnp.full((LANES,), _NEG, jnp.float32)

    for v in range(NV):
        obuf[pl.ds(v * LANES, LANES)] = neg

    def init_body(j, carry):
        pltpu.sync_copy(obuf, partials.at[j, s])
        return carry

    lax.fori_loop(0, NSEG, init_body, 0)

    def sum_body(j, tot):
        return tot + seg_len(j)

    total = lax.fori_loop(0, NSEG, sum_body, jnp.int32(0))
    u0 = (s * total) // NS
    u1 = ((s + 1) * total) // NS

    def seg_body(j, start):
        n = seg_len(j)
        b = BPC * c + j // 2
        h = j % 2
        lo = jnp.clip(u0 - start, 0, n)
        hi = jnp.clip(u1 - start, 0, n)

        @pl.when(lo < hi)
        def _process():
            c0 = lo // CHUNK
            nch = (hi + CHUNK - 1) // CHUNK - c0

            def src(ci):
                return padded_hbm.at[
                    b, pl.ds((c0 + ci) * CHUNK, CHUNK), pl.ds(h * DH, DH)
                ]

            pltpu.async_copy(src(0), buf0, sem0)

            def rows(ci, buf, accs):
                r0 = jnp.clip(lo - (c0 + ci) * CHUNK, 0, CHUNK)
                r1 = jnp.clip(hi - (c0 + ci) * CHUNK, 0, CHUNK)

                def row_body(r, accs):
                    return tuple(
                        jnp.maximum(accs[v], buf[r, pl.ds(v * LANES, LANES)])
                        for v in range(NV)
                    )

                return lax.fori_loop(r0, r1, row_body, accs)

            def pair_body(k, accs):
                ca = 2 * k
                cb = 2 * k + 1

                @pl.when(cb < nch)
                def _():
                    pltpu.async_copy(src(cb), buf1, sem1)

                pltpu.make_async_copy(src(ca), buf0, sem0).wait()
                accs = rows(ca, buf0, accs)

                @pl.when(ca + 2 < nch)
                def _():
                    pltpu.async_copy(src(ca + 2), buf0, sem0)

                @pl.when(cb < nch)
                def _():
                    pltpu.make_async_copy(src(cb), buf1, sem1).wait()

                accs = rows(cb, buf1, accs)
                return accs

            npairs = (nch + 1) // 2
            accs = lax.fori_loop(
                0, npairs, pair_body, tuple(neg for _ in range(NV))
            )

            for v in range(NV):
                obuf[pl.ds(v * LANES, LANES)] = accs[v]
            pltpu.sync_copy(obuf, partials.at[j, s])

        return start + n

    lax.fori_loop(0, NSEG, seg_body, jnp.int32(0))

    plsc.subcore_barrier()

    n_s = seg_len(s)
    b_s = BPC * c + s // 2
    h_s = s % 2
    pltpu.sync_copy(partials.at[s], cbuf)

    def comb_body(w, accs):
        return tuple(
            jnp.maximum(accs[v], cbuf[w, pl.ds(v * LANES, LANES)])
            for v in range(NV)
        )

    accs = lax.fori_loop(0, NS, comb_body, tuple(neg for _ in range(NV)))
    for v in range(NV):
        obuf[pl.ds(v * LANES, LANES)] = accs[v]

    @pl.when(n_s == 0)
    def _zeros():
        z = jnp.zeros((LANES,), jnp.float32)
        for v in range(NV):
            obuf[pl.ds(v * LANES, LANES)] = z

    pltpu.sync_copy(obuf, out_hbm.at[b_s, pl.ds(h_s * DH, DH)])


@jax.jit
def _pooled(padded, lens):
    mesh = plsc.VectorSubcoreMesh(
        core_axis_name="c", subcore_axis_name="s", num_cores=NC, num_subcores=NS
    )
    k = pl.kernel(
        _sc_body,
        out_type=jax.ShapeDtypeStruct((B, D), jnp.float32),
        mesh=mesh,
        scratch_types=[
            pltpu.VMEM((2 * B,), jnp.int32),
            pltpu.VMEM((CHUNK, DH), jnp.float32),
            pltpu.VMEM((CHUNK, DH), jnp.float32),
            pltpu.VMEM((DH,), jnp.float32),
            pltpu.VMEM((NS, DH), jnp.float32),
            pltpu.VMEM_SHARED((NSEG, NS, DH), jnp.float32),
            pltpu.SemaphoreType.DMA,
            pltpu.SemaphoreType.DMA,
        ],
    )
    return k(padded, lens)


BL = 256
NL = L // BL


def _tc_body(lens_ref, x_ref, o_ref):
    b = pl.program_id(0)
    l = pl.program_id(1)
    n = lens_ref[b]
    nb = (n + BL - 1) // BL

    @pl.when(jnp.logical_or(l == 0, l < nb))
    def _():
        def _masked(x):
            row = jax.lax.broadcasted_iota(jnp.int32, (1, BL, D), 1) + l * BL
            return jnp.max(jnp.where(row < n, x, -jnp.inf), axis=1, keepdims=True)

        def _plain(x):
            return jnp.max(x, axis=1, keepdims=True)

        m = jax.lax.cond((l + 1) * BL > n, _masked, _plain, x_ref[...])

        @pl.when(l == 0)
        def _():
            o_ref[...] = m

        @pl.when(l > 0)
        def _():
            o_ref[...] = jnp.maximum(o_ref[...], m)

    @pl.when(l == NL - 1)
    def _():
        o_ref[...] = jnp.where(n > 0, o_ref[...], 0.0)


@jax.jit
def _pooled_tc(padded, lens):
    return pl.pallas_call(
        _tc_body,
        grid_spec=pltpu.PrefetchScalarGridSpec(
            num_scalar_prefetch=1,
            grid=(B, NL),
            in_specs=[
                pl.BlockSpec(
                    (1, BL, D),
                    lambda b, l, lens: (b, 0, 0),
                ),
            ],
            out_specs=pl.BlockSpec((1, 1, D), lambda b, l, lens: (b, 0, 0)),
        ),
        out_shape=jax.ShapeDtypeStruct((B, 1, D), jnp.float32),
    )(lens, padded)[:, 0, :]


CH = 256


def _tc2_body(lens_ref, x_hbm, o_ref, buf0, buf1, sem0, sem1):
    b = pl.program_id(0)
    n = lens_ref[b]
    nch = (n + CH - 1) // CH

    def src(ci):
        return x_hbm.at[b, pl.ds(ci * CH, CH), :]

    def start(ci, buf, sem):
        pltpu.make_async_copy(src(ci), buf, sem).start()

    def wait(ci, buf, sem):
        pltpu.make_async_copy(src(ci), buf, sem).wait()

    o_ref[...] = jnp.full((1, 1, D), -jnp.inf, jnp.float32)

    @pl.when(nch > 0)
    def _():
        start(0, buf0, sem0)

    def compute(ci, buf):
        def _masked(x):
            row = jax.lax.broadcasted_iota(jnp.int32, (CH, D), 0) + ci * CH
            return jnp.max(jnp.where(row < n, x, -jnp.inf), axis=0, keepdims=True)

        def _plain(x):
            return jnp.max(x, axis=0, keepdims=True)

        m = jax.lax.cond((ci + 1) * CH > n, _masked, _plain, buf[...])
        o_ref[...] = jnp.maximum(o_ref[...], m[None])

    def chunk_body(ci, carry):
        @pl.when(ci % 2 == 0)
        def _():
            @pl.when(ci + 1 < nch)
            def _():
                start(ci + 1, buf1, sem1)

            wait(ci, buf0, sem0)
            compute(ci, buf0)

        @pl.when(ci % 2 == 1)
        def _():
            @pl.when(ci + 1 < nch)
            def _():
                start(ci + 1, buf0, sem0)

            wait(ci, buf1, sem1)
            compute(ci, buf1)

        return carry

    jax.lax.fori_loop(0, nch, chunk_body, 0)
    o_ref[...] = jnp.where(n > 0, o_ref[...], 0.0)


@jax.jit
def _pooled_tc2(padded, lens):
    return pl.pallas_call(
        _tc2_body,
        grid_spec=pltpu.PrefetchScalarGridSpec(
            num_scalar_prefetch=1,
            grid=(B,),
            in_specs=[pl.BlockSpec(memory_space=pl.ANY)],
            out_specs=pl.BlockSpec((1, 1, D), lambda b, lens: (b, 0, 0)),
            scratch_shapes=[
                pltpu.VMEM((CH, D), jnp.float32),
                pltpu.VMEM((CH, D), jnp.float32),
                pltpu.SemaphoreType.DMA,
                pltpu.SemaphoreType.DMA,
            ],
        ),
        out_shape=jax.ShapeDtypeStruct((B, 1, D), jnp.float32),
    )(lens, padded)[:, 0, :]


def kernel(sentence_embeddings_list, num_sentences, encoder_hidden_size):
    del encoder_hidden_size
    padded = sentence_embeddings_list.astype(jnp.float32)
    lens = num_sentences.astype(jnp.int32)
    return _pooled_tc2(padded, lens)


# speedup vs baseline: 1.1833x; 1.0480x over previous
import functools

import jax
import jax.numpy as jnp
from jax import lax
from jax.experimental import pallas as pl
from jax.experimental.pallas import tpu as pltpu
from jax.experimental.pallas import tpu_sc as plsc

B = 16
L = 4096
D = 1024

NC = 2
NS = 16
LANES = 16

BPC = B // NC
NSEG = 2 * BPC
CHUNK = 64
DH = D // 2
NV = DH // LANES

_NEG = float("-inf")


def _sc_body(padded_hbm, lens_hbm, out_hbm,
             lens_v, buf0, buf1, obuf, cbuf, partials, sem0, sem1):
    c = lax.axis_index("c")
    s = lax.axis_index("s")

    pltpu.sync_copy(lens_hbm, lens_v.at[pl.ds(0, B)])

    def seg_len(j):
        return lens_v[pl.ds(BPC * c + j // 2, LANES)][0]

    neg = jnp.full((LANES,), _NEG, jnp.float32)

    for v in range(NV):
        obuf[pl.ds(v * LANES, LANES)] = neg

    def init_body(j, carry):
        pltpu.sync_copy(obuf, partials.at[j, s])
        return carry

    lax.fori_loop(0, NSEG, init_body, 0)

    def sum_body(j, tot):
        return tot + seg_len(j)

    total = lax.fori_loop(0, NSEG, sum_body, jnp.int32(0))
    u0 = (s * total) // NS
    u1 = ((s + 1) * total) // NS

    def seg_body(j, start):
        n = seg_len(j)
        b = BPC * c + j // 2
        h = j % 2
        lo = jnp.clip(u0 - start, 0, n)
        hi = jnp.clip(u1 - start, 0, n)

        @pl.when(lo < hi)
        def _process():
            c0 = lo // CHUNK
            nch = (hi + CHUNK - 1) // CHUNK - c0

            def src(ci):
                return padded_hbm.at[
                    b, pl.ds((c0 + ci) * CHUNK, CHUNK), pl.ds(h * DH, DH)
                ]

            pltpu.async_copy(src(0), buf0, sem0)

            def rows(ci, buf, accs):
                r0 = jnp.clip(lo - (c0 + ci) * CHUNK, 0, CHUNK)
                r1 = jnp.clip(hi - (c0 + ci) * CHUNK, 0, CHUNK)

                def row_body(r, accs):
                    return tuple(
                        jnp.maximum(accs[v], buf[r, pl.ds(v * LANES, LANES)])
                        for v in range(NV)
                    )

                return lax.fori_loop(r0, r1, row_body, accs)

            def pair_body(k, accs):
                ca = 2 * k
                cb = 2 * k + 1

                @pl.when(cb < nch)
                def _():
                    pltpu.async_copy(src(cb), buf1, sem1)

                pltpu.make_async_copy(src(ca), buf0, sem0).wait()
                accs = rows(ca, buf0, accs)

                @pl.when(ca + 2 < nch)
                def _():
                    pltpu.async_copy(src(ca + 2), buf0, sem0)

                @pl.when(cb < nch)
                def _():
                    pltpu.make_async_copy(src(cb), buf1, sem1).wait()

                accs = rows(cb, buf1, accs)
                return accs

            npairs = (nch + 1) // 2
            accs = lax.fori_loop(
                0, npairs, pair_body, tuple(neg for _ in range(NV))
            )

            for v in range(NV):
                obuf[pl.ds(v * LANES, LANES)] = accs[v]
            pltpu.sync_copy(obuf, partials.at[j, s])

        return start + n

    lax.fori_loop(0, NSEG, seg_body, jnp.int32(0))

    plsc.subcore_barrier()

    n_s = seg_len(s)
    b_s = BPC * c + s // 2
    h_s = s % 2
    pltpu.sync_copy(partials.at[s], cbuf)

    def comb_body(w, accs):
        return tuple(
            jnp.maximum(accs[v], cbuf[w, pl.ds(v * LANES, LANES)])
            for v in range(NV)
        )

    accs = lax.fori_loop(0, NS, comb_body, tuple(neg for _ in range(NV)))
    for v in range(NV):
        obuf[pl.ds(v * LANES, LANES)] = accs[v]

    @pl.when(n_s == 0)
    def _zeros():
        z = jnp.zeros((LANES,), jnp.float32)
        for v in range(NV):
            obuf[pl.ds(v * LANES, LANES)] = z

    pltpu.sync_copy(obuf, out_hbm.at[b_s, pl.ds(h_s * DH, DH)])


@jax.jit
def _pooled(padded, lens):
    mesh = plsc.VectorSubcoreMesh(
        core_axis_name="c", subcore_axis_name="s", num_cores=NC, num_subcores=NS
    )
    k = pl.kernel(
        _sc_body,
        out_type=jax.ShapeDtypeStruct((B, D), jnp.float32),
        mesh=mesh,
        scratch_types=[
            pltpu.VMEM((2 * B,), jnp.int32),
            pltpu.VMEM((CHUNK, DH), jnp.float32),
            pltpu.VMEM((CHUNK, DH), jnp.float32),
            pltpu.VMEM((DH,), jnp.float32),
            pltpu.VMEM((NS, DH), jnp.float32),
            pltpu.VMEM_SHARED((NSEG, NS, DH), jnp.float32),
            pltpu.SemaphoreType.DMA,
            pltpu.SemaphoreType.DMA,
        ],
    )
    return k(padded, lens)


BL = 256
NL = L // BL


def _tc_body(lens_ref, x_ref, o_ref):
    b = pl.program_id(0)
    l = pl.program_id(1)
    n = lens_ref[b]
    nb = (n + BL - 1) // BL

    @pl.when(jnp.logical_or(l == 0, l < nb))
    def _():
        def _masked(x):
            row = jax.lax.broadcasted_iota(jnp.int32, (1, BL, D), 1) + l * BL
            return jnp.max(jnp.where(row < n, x, -jnp.inf), axis=1, keepdims=True)

        def _plain(x):
            return jnp.max(x, axis=1, keepdims=True)

        m = jax.lax.cond((l + 1) * BL > n, _masked, _plain, x_ref[...])

        @pl.when(l == 0)
        def _():
            o_ref[...] = m

        @pl.when(l > 0)
        def _():
            o_ref[...] = jnp.maximum(o_ref[...], m)

    @pl.when(l == NL - 1)
    def _():
        o_ref[...] = jnp.where(n > 0, o_ref[...], 0.0)


@jax.jit
def _pooled_tc(padded, lens):
    return pl.pallas_call(
        _tc_body,
        grid_spec=pltpu.PrefetchScalarGridSpec(
            num_scalar_prefetch=1,
            grid=(B, NL),
            in_specs=[
                pl.BlockSpec(
                    (1, BL, D),
                    lambda b, l, lens: (b, 0, 0),
                ),
            ],
            out_specs=pl.BlockSpec((1, 1, D), lambda b, l, lens: (b, 0, 0)),
        ),
        out_shape=jax.ShapeDtypeStruct((B, 1, D), jnp.float32),
    )(lens, padded)[:, 0, :]


CH = 256


def _tc2_body(lens_ref, x_hbm, o_ref, buf0, buf1, sem0, sem1):
    b = pl.program_id(0)
    n = lens_ref[b]
    nch = (n + CH - 1) // CH

    def src(ci):
        return x_hbm.at[b, pl.ds(ci * CH, CH), :]

    def start(ci, buf, sem):
        pltpu.make_async_copy(src(ci), buf, sem).start()

    def wait(ci, buf, sem):
        pltpu.make_async_copy(src(ci), buf, sem).wait()

    o_ref[...] = jnp.full((1, 1, D), -jnp.inf, jnp.float32)

    @pl.when(nch > 0)
    def _():
        start(0, buf0, sem0)

    def compute(ci, buf):
        boundary = (ci + 1) * CH > n

        @pl.when(boundary)
        def _():
            row = jax.lax.broadcasted_iota(jnp.int32, (CH, D), 0) + ci * CH
            m = jnp.max(
                jnp.where(row < n, buf[...], -jnp.inf), axis=0, keepdims=True
            )
            o_ref[...] = jnp.maximum(o_ref[...], m[None])

        @pl.when(jnp.logical_not(boundary))
        def _():
            m = jnp.max(buf[...], axis=0, keepdims=True)
            o_ref[...] = jnp.maximum(o_ref[...], m[None])

    def chunk_body(ci, carry):
        @pl.when(ci % 2 == 0)
        def _():
            @pl.when(ci + 1 < nch)
            def _():
                start(ci + 1, buf1, sem1)

            wait(ci, buf0, sem0)
            compute(ci, buf0)

        @pl.when(ci % 2 == 1)
        def _():
            @pl.when(ci + 1 < nch)
            def _():
                start(ci + 1, buf0, sem0)

            wait(ci, buf1, sem1)
            compute(ci, buf1)

        return carry

    jax.lax.fori_loop(0, nch, chunk_body, 0)
    o_ref[...] = jnp.where(n > 0, o_ref[...], 0.0)


@jax.jit
def _pooled_tc2(padded, lens):
    return pl.pallas_call(
        _tc2_body,
        grid_spec=pltpu.PrefetchScalarGridSpec(
            num_scalar_prefetch=1,
            grid=(B,),
            in_specs=[pl.BlockSpec(memory_space=pl.ANY)],
            out_specs=pl.BlockSpec((1, 1, D), lambda b, lens: (b, 0, 0)),
            scratch_shapes=[
                pltpu.VMEM((CH, D), jnp.float32),
                pltpu.VMEM((CH, D), jnp.float32),
                pltpu.SemaphoreType.DMA,
                pltpu.SemaphoreType.DMA,
            ],
        ),
        out_shape=jax.ShapeDtypeStruct((B, 1, D), jnp.float32),
    )(lens, padded)[:, 0, :]


def kernel(sentence_embeddings_list, num_sentences, encoder_hidden_size):
    del encoder_hidden_size
    padded = sentence_embeddings_list.astype(jnp.float32)
    lens = num_sentences.astype(jnp.int32)
    return _pooled_tc2(padded, lens)


# speedup vs baseline: 1.7464x; 1.4759x over previous
import functools

import jax
import jax.numpy as jnp
from jax import lax
from jax.experimental import pallas as pl
from jax.experimental.pallas import tpu as pltpu
from jax.experimental.pallas import tpu_sc as plsc

B = 16
L = 4096
D = 1024

NC = 2
NS = 16
LANES = 16

BPC = B // NC
NSEG = 2 * BPC
CHUNK = 64
DH = D // 2
NV = DH // LANES

_NEG = float("-inf")


def _sc_body(padded_hbm, lens_hbm, out_hbm,
             lens_v, buf0, buf1, obuf, cbuf, partials, sem0, sem1):
    c = lax.axis_index("c")
    s = lax.axis_index("s")

    pltpu.sync_copy(lens_hbm, lens_v.at[pl.ds(0, B)])

    def seg_len(j):
        return lens_v[pl.ds(BPC * c + j // 2, LANES)][0]

    neg = jnp.full((LANES,), _NEG, jnp.float32)

    for v in range(NV):
        obuf[pl.ds(v * LANES, LANES)] = neg

    def init_body(j, carry):
        pltpu.sync_copy(obuf, partials.at[j, s])
        return carry

    lax.fori_loop(0, NSEG, init_body, 0)

    def sum_body(j, tot):
        return tot + seg_len(j)

    total = lax.fori_loop(0, NSEG, sum_body, jnp.int32(0))
    u0 = (s * total) // NS
    u1 = ((s + 1) * total) // NS

    def seg_body(j, start):
        n = seg_len(j)
        b = BPC * c + j // 2
        h = j % 2
        lo = jnp.clip(u0 - start, 0, n)
        hi = jnp.clip(u1 - start, 0, n)

        @pl.when(lo < hi)
        def _process():
            c0 = lo // CHUNK
            nch = (hi + CHUNK - 1) // CHUNK - c0

            def src(ci):
                return padded_hbm.at[
                    b, pl.ds((c0 + ci) * CHUNK, CHUNK), pl.ds(h * DH, DH)
                ]

            pltpu.async_copy(src(0), buf0, sem0)

            def rows(ci, buf, accs):
                r0 = jnp.clip(lo - (c0 + ci) * CHUNK, 0, CHUNK)
                r1 = jnp.clip(hi - (c0 + ci) * CHUNK, 0, CHUNK)

                def row_body(r, accs):
                    return tuple(
                        jnp.maximum(accs[v], buf[r, pl.ds(v * LANES, LANES)])
                        for v in range(NV)
                    )

                return lax.fori_loop(r0, r1, row_body, accs)

            def pair_body(k, accs):
                ca = 2 * k
                cb = 2 * k + 1

                @pl.when(cb < nch)
                def _():
                    pltpu.async_copy(src(cb), buf1, sem1)

                pltpu.make_async_copy(src(ca), buf0, sem0).wait()
                accs = rows(ca, buf0, accs)

                @pl.when(ca + 2 < nch)
                def _():
                    pltpu.async_copy(src(ca + 2), buf0, sem0)

                @pl.when(cb < nch)
                def _():
                    pltpu.make_async_copy(src(cb), buf1, sem1).wait()

                accs = rows(cb, buf1, accs)
                return accs

            npairs = (nch + 1) // 2
            accs = lax.fori_loop(
                0, npairs, pair_body, tuple(neg for _ in range(NV))
            )

            for v in range(NV):
                obuf[pl.ds(v * LANES, LANES)] = accs[v]
            pltpu.sync_copy(obuf, partials.at[j, s])

        return start + n

    lax.fori_loop(0, NSEG, seg_body, jnp.int32(0))

    plsc.subcore_barrier()

    n_s = seg_len(s)
    b_s = BPC * c + s // 2
    h_s = s % 2
    pltpu.sync_copy(partials.at[s], cbuf)

    def comb_body(w, accs):
        return tuple(
            jnp.maximum(accs[v], cbuf[w, pl.ds(v * LANES, LANES)])
            for v in range(NV)
        )

    accs = lax.fori_loop(0, NS, comb_body, tuple(neg for _ in range(NV)))
    for v in range(NV):
        obuf[pl.ds(v * LANES, LANES)] = accs[v]

    @pl.when(n_s == 0)
    def _zeros():
        z = jnp.zeros((LANES,), jnp.float32)
        for v in range(NV):
            obuf[pl.ds(v * LANES, LANES)] = z

    pltpu.sync_copy(obuf, out_hbm.at[b_s, pl.ds(h_s * DH, DH)])


@jax.jit
def _pooled(padded, lens):
    mesh = plsc.VectorSubcoreMesh(
        core_axis_name="c", subcore_axis_name="s", num_cores=NC, num_subcores=NS
    )
    k = pl.kernel(
        _sc_body,
        out_type=jax.ShapeDtypeStruct((B, D), jnp.float32),
        mesh=mesh,
        scratch_types=[
            pltpu.VMEM((2 * B,), jnp.int32),
            pltpu.VMEM((CHUNK, DH), jnp.float32),
            pltpu.VMEM((CHUNK, DH), jnp.float32),
            pltpu.VMEM((DH,), jnp.float32),
            pltpu.VMEM((NS, DH), jnp.float32),
            pltpu.VMEM_SHARED((NSEG, NS, DH), jnp.float32),
            pltpu.SemaphoreType.DMA,
            pltpu.SemaphoreType.DMA,
        ],
    )
    return k(padded, lens)


BL = 256
NL = L // BL


def _tc_body(lens_ref, x_ref, o_ref):
    b = pl.program_id(0)
    l = pl.program_id(1)
    n = lens_ref[b]
    nb = (n + BL - 1) // BL

    @pl.when(jnp.logical_or(l == 0, l < nb))
    def _():
        def _masked(x):
            row = jax.lax.broadcasted_iota(jnp.int32, (1, BL, D), 1) + l * BL
            return jnp.max(jnp.where(row < n, x, -jnp.inf), axis=1, keepdims=True)

        def _plain(x):
            return jnp.max(x, axis=1, keepdims=True)

        m = jax.lax.cond((l + 1) * BL > n, _masked, _plain, x_ref[...])

        @pl.when(l == 0)
        def _():
            o_ref[...] = m

        @pl.when(l > 0)
        def _():
            o_ref[...] = jnp.maximum(o_ref[...], m)

    @pl.when(l == NL - 1)
    def _():
        o_ref[...] = jnp.where(n > 0, o_ref[...], 0.0)


@jax.jit
def _pooled_tc(padded, lens):
    return pl.pallas_call(
        _tc_body,
        grid_spec=pltpu.PrefetchScalarGridSpec(
            num_scalar_prefetch=1,
            grid=(B, NL),
            in_specs=[
                pl.BlockSpec(
                    (1, BL, D),
                    lambda b, l, lens: (b, 0, 0),
                ),
            ],
            out_specs=pl.BlockSpec((1, 1, D), lambda b, l, lens: (b, 0, 0)),
        ),
        out_shape=jax.ShapeDtypeStruct((B, 1, D), jnp.float32),
    )(lens, padded)[:, 0, :]


CH = 256


NBUF = 4


def _tc2_body(lens_ref, x_hbm, o_ref, *scratch):
    bufs = scratch[:NBUF]
    sems = scratch[NBUF:]
    b = pl.program_id(0)
    n = lens_ref[b]
    nch = (n + CH - 1) // CH

    def src(ci):
        return x_hbm.at[b, pl.ds(ci * CH, CH), :]

    def start(ci, buf, sem):
        pltpu.make_async_copy(src(ci), buf, sem).start()

    def wait(ci, buf, sem):
        pltpu.make_async_copy(src(ci), buf, sem).wait()

    o_ref[...] = jnp.full((1, 1, D), -jnp.inf, jnp.float32)

    for k in range(NBUF - 1):
        @pl.when(k < nch)
        def _(k=k):
            start(k, bufs[k], sems[k])

    def compute(ci, buf):
        boundary = (ci + 1) * CH > n

        @pl.when(boundary)
        def _():
            row = jax.lax.broadcasted_iota(jnp.int32, (CH, D), 0) + ci * CH
            m = jnp.max(
                jnp.where(row < n, buf[...], -jnp.inf), axis=0, keepdims=True
            )
            o_ref[...] = jnp.maximum(o_ref[...], m[None])

        @pl.when(jnp.logical_not(boundary))
        def _():
            m = jnp.max(buf[...], axis=0, keepdims=True)
            o_ref[...] = jnp.maximum(o_ref[...], m[None])

    def chunk_body(ci, carry):
        for p in range(NBUF):
            @pl.when(ci % NBUF == p)
            def _(p=p):
                @pl.when(ci + NBUF - 1 < nch)
                def _():
                    q = (p + NBUF - 1) % NBUF
                    start(ci + NBUF - 1, bufs[q], sems[q])

                wait(ci, bufs[p], sems[p])
                compute(ci, bufs[p])

        return carry

    jax.lax.fori_loop(0, nch, chunk_body, 0)
    o_ref[...] = jnp.where(n > 0, o_ref[...], 0.0)


@jax.jit
def _pooled_tc2(padded, lens):
    return pl.pallas_call(
        _tc2_body,
        grid_spec=pltpu.PrefetchScalarGridSpec(
            num_scalar_prefetch=1,
            grid=(B,),
            in_specs=[pl.BlockSpec(memory_space=pl.ANY)],
            out_specs=pl.BlockSpec((1, 1, D), lambda b, lens: (b, 0, 0)),
            scratch_shapes=(
                [pltpu.VMEM((CH, D), jnp.float32)] * NBUF
                + [pltpu.SemaphoreType.DMA] * NBUF
            ),
        ),
        out_shape=jax.ShapeDtypeStruct((B, 1, D), jnp.float32),
    )(lens, padded)[:, 0, :]


def kernel(sentence_embeddings_list, num_sentences, encoder_hidden_size):
    del encoder_hidden_size
    padded = sentence_embeddings_list.astype(jnp.float32)
    lens = num_sentences.astype(jnp.int32)
    return _pooled_tc2(padded, lens)


# speedup vs baseline: 1.8877x; 1.0809x over previous
import functools

import jax
import jax.numpy as jnp
from jax import lax
from jax.experimental import pallas as pl
from jax.experimental.pallas import tpu as pltpu
from jax.experimental.pallas import tpu_sc as plsc

B = 16
L = 4096
D = 1024

NC = 2
NS = 16
LANES = 16

BPC = B // NC
NSEG = 2 * BPC
CHUNK = 64
DH = D // 2
NV = DH // LANES

_NEG = float("-inf")


def _sc_body(padded_hbm, lens_hbm, out_hbm,
             lens_v, buf0, buf1, obuf, cbuf, partials, sem0, sem1):
    c = lax.axis_index("c")
    s = lax.axis_index("s")

    pltpu.sync_copy(lens_hbm, lens_v.at[pl.ds(0, B)])

    def seg_len(j):
        return lens_v[pl.ds(BPC * c + j // 2, LANES)][0]

    neg = jnp.full((LANES,), _NEG, jnp.float32)

    for v in range(NV):
        obuf[pl.ds(v * LANES, LANES)] = neg

    def init_body(j, carry):
        pltpu.sync_copy(obuf, partials.at[j, s])
        return carry

    lax.fori_loop(0, NSEG, init_body, 0)

    def sum_body(j, tot):
        return tot + seg_len(j)

    total = lax.fori_loop(0, NSEG, sum_body, jnp.int32(0))
    u0 = (s * total) // NS
    u1 = ((s + 1) * total) // NS

    def seg_body(j, start):
        n = seg_len(j)
        b = BPC * c + j // 2
        h = j % 2
        lo = jnp.clip(u0 - start, 0, n)
        hi = jnp.clip(u1 - start, 0, n)

        @pl.when(lo < hi)
        def _process():
            c0 = lo // CHUNK
            nch = (hi + CHUNK - 1) // CHUNK - c0

            def src(ci):
                return padded_hbm.at[
                    b, pl.ds((c0 + ci) * CHUNK, CHUNK), pl.ds(h * DH, DH)
                ]

            pltpu.async_copy(src(0), buf0, sem0)

            def rows(ci, buf, accs):
                r0 = jnp.clip(lo - (c0 + ci) * CHUNK, 0, CHUNK)
                r1 = jnp.clip(hi - (c0 + ci) * CHUNK, 0, CHUNK)

                def row_body(r, accs):
                    return tuple(
                        jnp.maximum(accs[v], buf[r, pl.ds(v * LANES, LANES)])
                        for v in range(NV)
                    )

                return lax.fori_loop(r0, r1, row_body, accs)

            def pair_body(k, accs):
                ca = 2 * k
                cb = 2 * k + 1

                @pl.when(cb < nch)
                def _():
                    pltpu.async_copy(src(cb), buf1, sem1)

                pltpu.make_async_copy(src(ca), buf0, sem0).wait()
                accs = rows(ca, buf0, accs)

                @pl.when(ca + 2 < nch)
                def _():
                    pltpu.async_copy(src(ca + 2), buf0, sem0)

                @pl.when(cb < nch)
                def _():
                    pltpu.make_async_copy(src(cb), buf1, sem1).wait()

                accs = rows(cb, buf1, accs)
                return accs

            npairs = (nch + 1) // 2
            accs = lax.fori_loop(
                0, npairs, pair_body, tuple(neg for _ in range(NV))
            )

            for v in range(NV):
                obuf[pl.ds(v * LANES, LANES)] = accs[v]
            pltpu.sync_copy(obuf, partials.at[j, s])

        return start + n

    lax.fori_loop(0, NSEG, seg_body, jnp.int32(0))

    plsc.subcore_barrier()

    n_s = seg_len(s)
    b_s = BPC * c + s // 2
    h_s = s % 2
    pltpu.sync_copy(partials.at[s], cbuf)

    def comb_body(w, accs):
        return tuple(
            jnp.maximum(accs[v], cbuf[w, pl.ds(v * LANES, LANES)])
            for v in range(NV)
        )

    accs = lax.fori_loop(0, NS, comb_body, tuple(neg for _ in range(NV)))
    for v in range(NV):
        obuf[pl.ds(v * LANES, LANES)] = accs[v]

    @pl.when(n_s == 0)
    def _zeros():
        z = jnp.zeros((LANES,), jnp.float32)
        for v in range(NV):
            obuf[pl.ds(v * LANES, LANES)] = z

    pltpu.sync_copy(obuf, out_hbm.at[b_s, pl.ds(h_s * DH, DH)])


@jax.jit
def _pooled(padded, lens):
    mesh = plsc.VectorSubcoreMesh(
        core_axis_name="c", subcore_axis_name="s", num_cores=NC, num_subcores=NS
    )
    k = pl.kernel(
        _sc_body,
        out_type=jax.ShapeDtypeStruct((B, D), jnp.float32),
        mesh=mesh,
        scratch_types=[
            pltpu.VMEM((2 * B,), jnp.int32),
            pltpu.VMEM((CHUNK, DH), jnp.float32),
            pltpu.VMEM((CHUNK, DH), jnp.float32),
            pltpu.VMEM((DH,), jnp.float32),
            pltpu.VMEM((NS, DH), jnp.float32),
            pltpu.VMEM_SHARED((NSEG, NS, DH), jnp.float32),
            pltpu.SemaphoreType.DMA,
            pltpu.SemaphoreType.DMA,
        ],
    )
    return k(padded, lens)


BL = 256
NL = L // BL


def _tc_body(lens_ref, x_ref, o_ref):
    b = pl.program_id(0)
    l = pl.program_id(1)
    n = lens_ref[b]
    nb = (n + BL - 1) // BL

    @pl.when(jnp.logical_or(l == 0, l < nb))
    def _():
        def _masked(x):
            row = jax.lax.broadcasted_iota(jnp.int32, (1, BL, D), 1) + l * BL
            return jnp.max(jnp.where(row < n, x, -jnp.inf), axis=1, keepdims=True)

        def _plain(x):
            return jnp.max(x, axis=1, keepdims=True)

        m = jax.lax.cond((l + 1) * BL > n, _masked, _plain, x_ref[...])

        @pl.when(l == 0)
        def _():
            o_ref[...] = m

        @pl.when(l > 0)
        def _():
            o_ref[...] = jnp.maximum(o_ref[...], m)

    @pl.when(l == NL - 1)
    def _():
        o_ref[...] = jnp.where(n > 0, o_ref[...], 0.0)


@jax.jit
def _pooled_tc(padded, lens):
    return pl.pallas_call(
        _tc_body,
        grid_spec=pltpu.PrefetchScalarGridSpec(
            num_scalar_prefetch=1,
            grid=(B, NL),
            in_specs=[
                pl.BlockSpec(
                    (1, BL, D),
                    lambda b, l, lens: (b, 0, 0),
                ),
            ],
            out_specs=pl.BlockSpec((1, 1, D), lambda b, l, lens: (b, 0, 0)),
        ),
        out_shape=jax.ShapeDtypeStruct((B, 1, D), jnp.float32),
    )(lens, padded)[:, 0, :]


CH = 256


NBUF = 8


def _tc2_body(lens_ref, x_hbm, o_ref, *scratch):
    bufs = scratch[:NBUF]
    sems = scratch[NBUF:]
    b = pl.program_id(0)
    n = lens_ref[b]
    nch = (n + CH - 1) // CH

    def src(ci):
        return x_hbm.at[b, pl.ds(ci * CH, CH), :]

    def start(ci, buf, sem):
        pltpu.make_async_copy(src(ci), buf, sem).start()

    def wait(ci, buf, sem):
        pltpu.make_async_copy(src(ci), buf, sem).wait()

    o_ref[...] = jnp.full((1, 1, D), -jnp.inf, jnp.float32)

    for k in range(NBUF - 1):
        @pl.when(k < nch)
        def _(k=k):
            start(k, bufs[k], sems[k])

    def compute(ci, buf):
        boundary = (ci + 1) * CH > n

        @pl.when(boundary)
        def _():
            row = jax.lax.broadcasted_iota(jnp.int32, (CH, D), 0) + ci * CH
            m = jnp.max(
                jnp.where(row < n, buf[...], -jnp.inf), axis=0, keepdims=True
            )
            o_ref[...] = jnp.maximum(o_ref[...], m[None])

        @pl.when(jnp.logical_not(boundary))
        def _():
            m = jnp.max(buf[...], axis=0, keepdims=True)
            o_ref[...] = jnp.maximum(o_ref[...], m[None])

    def chunk_body(ci, carry):
        for p in range(NBUF):
            @pl.when(ci % NBUF == p)
            def _(p=p):
                @pl.when(ci + NBUF - 1 < nch)
                def _():
                    q = (p + NBUF - 1) % NBUF
                    start(ci + NBUF - 1, bufs[q], sems[q])

                wait(ci, bufs[p], sems[p])
                compute(ci, bufs[p])

        return carry

    jax.lax.fori_loop(0, nch, chunk_body, 0)
    o_ref[...] = jnp.where(n > 0, o_ref[...], 0.0)


@jax.jit
def _pooled_tc2(padded, lens):
    return pl.pallas_call(
        _tc2_body,
        grid_spec=pltpu.PrefetchScalarGridSpec(
            num_scalar_prefetch=1,
            grid=(B,),
            in_specs=[pl.BlockSpec(memory_space=pl.ANY)],
            out_specs=pl.BlockSpec((1, 1, D), lambda b, lens: (b, 0, 0)),
            scratch_shapes=(
                [pltpu.VMEM((CH, D), jnp.float32)] * NBUF
                + [pltpu.SemaphoreType.DMA] * NBUF
            ),
        ),
        out_shape=jax.ShapeDtypeStruct((B, 1, D), jnp.float32),
    )(lens, padded)[:, 0, :]


def kernel(sentence_embeddings_list, num_sentences, encoder_hidden_size):
    del encoder_hidden_size
    padded = sentence_embeddings_list.astype(jnp.float32)
    lens = num_sentences.astype(jnp.int32)
    return _pooled_tc2(padded, lens)
